# baseline (speedup 1.0000x reference)
import functools

import jax
import jax.numpy as jnp
from jax import lax
from jax.experimental import pallas as pl
from jax.experimental.pallas import tpu as pltpu
from jax.experimental.pallas import tpu_sc as plsc

N = 10000
D = 128
E = 320000
NC = 2
NS = 16
NW = NC * NS
BLK = 128
NB = E // BLK
MAXB = 80
NBPAD = NW * MAXB
EPAD = NBPAD * BLK
HB = MAXB // 2
NPAD = 10240
ROWS_T = NPAD // NS
DEGP = 10240
DEG_T = DEGP // NS

_mesh = plsc.VectorSubcoreMesh(core_axis_name="c", subcore_axis_name="s")


def _tile_range(c, s, maxb, num_blocks):
    wid = c * NS + s
    start = maxb * wid
    nb = jnp.clip(num_blocks - start, 0, maxb)
    return start, nb


def _deg_body(dstb, deg_out, dst_v, ones_v, zb_v, acc, sem_a, sem_b):
    c = lax.axis_index("c")
    s = lax.axis_index("s")
    start, nb = _tile_range(c, s, MAXB, NB)

    pltpu.sync_copy(dstb.at[pl.ds(start, MAXB)], dst_v)

    def zero_body(i, _):
        zb_v[pl.ds(i * 16, 16)] = jnp.zeros((16,), jnp.float32)
        return 0

    lax.fori_loop(0, DEG_T // 16, zero_body, 0)
    for j in range(BLK // 16):
        ones_v[pl.ds(j * 16, 16)] = jnp.ones((16,), jnp.float32)
    pltpu.sync_copy(zb_v, acc.at[pl.ds(DEG_T * s, DEG_T)])
    plsc.subcore_barrier()

    def add_start(b, sem):
        pltpu.async_copy(ones_v, acc.at[dst_v.at[b]], sem, add=True)

    def add_wait(b, sem):
        pltpu.make_async_copy(ones_v, acc.at[dst_v.at[b]], sem).wait()

    def body(i, _):
        b0 = 2 * i

        @pl.when(b0 < nb)
        def _():
            @pl.when(b0 >= 2)
            def _():
                add_wait(b0 - 2, sem_a)

            add_start(b0, sem_a)

            @pl.when(b0 >= 2)
            def _():
                add_wait(b0 - 1, sem_b)

            add_start(b0 + 1, sem_b)

        return 0

    lax.fori_loop(0, MAXB // 2, body, 0)
    add_wait(nb - 2, sem_a)
    add_wait(nb - 1, sem_b)
    plsc.subcore_barrier()
    pltpu.sync_copy(acc.at[pl.ds(DEG_T * s, DEG_T)],
                    deg_out.at[c, pl.ds(DEG_T * s, DEG_T)])


_deg_kernel = functools.partial(
    pl.kernel,
    out_type=jax.ShapeDtypeStruct((NC, DEGP), jnp.float32),
    mesh=_mesh,
    scratch_types=[
        pltpu.VMEM((MAXB, BLK), jnp.int32),
        pltpu.VMEM((BLK,), jnp.float32),
        pltpu.VMEM((DEG_T,), jnp.float32),
        pltpu.VMEM_SHARED((DEGP,), jnp.float32),
        pltpu.SemaphoreType.DMA,
        pltpu.SemaphoreType.DMA,
    ],
)(_deg_body)


def _seg_body(srcb, dstb, xs, znd, sum_out,
              src_v, dst_v, rows_a, rows_b, acc,
              sem_a, sem_b, sem_sa, sem_sb):
    c = lax.axis_index("c")
    s = lax.axis_index("s")
    start, nb = _tile_range(c, s, MAXB, NB)

    pltpu.sync_copy(znd.at[pl.ds(ROWS_T * s, ROWS_T)],
                    acc.at[pl.ds(ROWS_T * s, ROWS_T)])
    plsc.subcore_barrier()

    def g_a(b):
        return pltpu.make_async_copy(xs.at[src_v.at[b]], rows_a, sem_a)

    def g_b(b):
        return pltpu.make_async_copy(xs.at[src_v.at[b]], rows_b, sem_b)

    def s_a_start(b):
        pltpu.async_copy(rows_a, acc.at[dst_v.at[b]], sem_sa, add=True)

    def s_a_wait(b):
        pltpu.make_async_copy(rows_a, acc.at[dst_v.at[b]], sem_sa).wait()

    def s_b_start(b):
        pltpu.async_copy(rows_b, acc.at[dst_v.at[b]], sem_sb, add=True)

    def s_b_wait(b):
        pltpu.make_async_copy(rows_b, acc.at[dst_v.at[b]], sem_sb).wait()

    for h in range(MAXB // HB):
        nb_h = jnp.clip(nb - h * HB, 0, HB)

        @pl.when(nb_h > 0)
        def _():
            pltpu.sync_copy(srcb.at[pl.ds(start + h * HB, HB)], src_v)
            pltpu.sync_copy(dstb.at[pl.ds(start + h * HB, HB)], dst_v)
            g_a(0).start()
            g_b(1).start()

            def body(i, _):
                b0 = 2 * i

                @pl.when(b0 < nb_h)
                def _():
                    g_a(b0).wait()
                    s_a_start(b0)
                    g_b(b0 + 1).wait()
                    s_b_start(b0 + 1)

                    @pl.when(b0 + 2 < nb_h)
                    def _():
                        s_a_wait(b0)
                        g_a(b0 + 2).start()

                    @pl.when(b0 + 3 < nb_h)
                    def _():
                        s_b_wait(b0 + 1)
                        g_b(b0 + 3).start()

                return 0

            lax.fori_loop(0, HB // 2, body, 0)
            s_a_wait(nb_h - 2)
            s_b_wait(nb_h - 1)

    plsc.subcore_barrier()
    pltpu.sync_copy(acc.at[pl.ds(ROWS_T * s, ROWS_T)],
                    sum_out.at[c, pl.ds(ROWS_T * s, ROWS_T)])


_seg_kernel = functools.partial(
    pl.kernel,
    out_type=jax.ShapeDtypeStruct((NC, NPAD, D), jnp.float32),
    mesh=_mesh,
    scratch_types=[
        pltpu.VMEM((HB, BLK), jnp.int32),
        pltpu.VMEM((HB, BLK), jnp.int32),
        pltpu.VMEM((BLK, D), jnp.float32),
        pltpu.VMEM((BLK, D), jnp.float32),
        pltpu.VMEM_SHARED((NPAD, D), jnp.float32),
        pltpu.SemaphoreType.DMA,
        pltpu.SemaphoreType.DMA,
        pltpu.SemaphoreType.DMA,
        pltpu.SemaphoreType.DMA,
    ],
)(_seg_body)


_XS_ROWS = 2000


def _xs_body(d0_ref, d1_ref, x_ref, o_ref):
    dis = lax.rsqrt(d0_ref[...] + d1_ref[...] + 1.0)
    o_ref[...] = x_ref[...] * dis


def _xs_call(d0, d1, x):
    g = N // _XS_ROWS
    return pl.pallas_call(
        _xs_body,
        grid=(g,),
        in_specs=[
            pl.BlockSpec((_XS_ROWS, 1), lambda i: (i, 0)),
            pl.BlockSpec((_XS_ROWS, 1), lambda i: (i, 0)),
            pl.BlockSpec((_XS_ROWS, D), lambda i: (i, 0)),
        ],
        out_specs=pl.BlockSpec((_XS_ROWS, D), lambda i: (i, 0)),
        out_shape=jax.ShapeDtypeStruct((N, D), jnp.float32),
    )(d0, d1, x)


_HD_ROWS = 2000
MID = 256
PMID = 8


def _dot(a, b):
    return jnp.dot(a, b, preferred_element_type=jnp.float32)


def _head_body(d0_ref, d1_ref, sp_ref, xs_ref, x_ref,
               w1_ref, b1_ref, l1_ref, l1b_ref, l2_ref, l2b_ref,
               l3_ref, l3b_ref, wp_ref, bp_ref, l1p_ref, l1pb_ref, bil_ref,
               acc_ref, yp_ref, xp_ref):
    dis = lax.rsqrt(d0_ref[...] + d1_ref[...] + 1.0)
    x = x_ref[...]
    agg = dis * (sp_ref[0] + sp_ref[1] + xs_ref[...])

    o1 = jax.nn.relu(_dot(agg, w1_ref[...]) + b1_ref[...]) + x
    h = jax.nn.relu(_dot(o1, l1_ref[...]) + l1b_ref[...])
    h = jax.nn.relu(_dot(h, l2_ref[...]) + l2b_ref[...])
    acc_ref[...] = _dot(h, l3_ref[...]) + l3b_ref[...]

    op = jax.nn.relu(_dot(agg, wp_ref[...]) + bp_ref[...]) + x
    xp = jax.nn.relu(_dot(op, l1p_ref[...]) + l1pb_ref[...])
    yp_ref[...] = _dot(xp, bil_ref[...])
    xp_ref[...] = xp


def _head_call(d0, d1, s_parts, xs, x, w1, b1, l1, l1b, l2, l2b, l3, l3b,
               wp, bp, l1p, l1pb, bil):
    g = N // _HD_ROWS
    row = lambda i: (i, 0)
    full = lambda i: (0, 0)
    return pl.pallas_call(
        _head_body,
        grid=(g,),
        in_specs=[
            pl.BlockSpec((_HD_ROWS, 1), row),
            pl.BlockSpec((_HD_ROWS, 1), row),
            pl.BlockSpec((NC, _HD_ROWS, D), lambda i: (0, i, 0)),
            pl.BlockSpec((_HD_ROWS, D), row),
            pl.BlockSpec((_HD_ROWS, D), row),
            pl.BlockSpec((D, D), full),
            pl.BlockSpec((1, D), full),
            pl.BlockSpec((D, MID), full),
            pl.BlockSpec((1, MID), full),
            pl.BlockSpec((MID, MID), full),
            pl.BlockSpec((1, MID), full),
            pl.BlockSpec((MID, 1), full),
            pl.BlockSpec((1, 1), full),
            pl.BlockSpec((D, D), full),
            pl.BlockSpec((1, D), full),
            pl.BlockSpec((D, PMID), full),
            pl.BlockSpec((1, PMID), full),
            pl.BlockSpec((PMID, PMID), full),
        ],
        out_specs=[
            pl.BlockSpec((_HD_ROWS, 1), row),
            pl.BlockSpec((_HD_ROWS, PMID), row),
            pl.BlockSpec((_HD_ROWS, PMID), row),
        ],
        out_shape=[
            jax.ShapeDtypeStruct((N, 1), jnp.float32),
            jax.ShapeDtypeStruct((N, PMID), jnp.float32),
            jax.ShapeDtypeStruct((N, PMID), jnp.float32),
        ],
    )(d0, d1, s_parts, xs, x, w1, b1, l1, l1b, l2, l2b, l3, l3b,
      wp, bp, l1p, l1pb, bil)


_PR_ROWS = 400


def _price_body(yp_ref, xpt_ref, pr_ref, o_ref):
    o_ref[...] = jnp.dot(yp_ref[...], xpt_ref[...],
                         preferred_element_type=jnp.float32) + pr_ref[...]


def _price_call(yp, xpt, pr):
    g = N // _PR_ROWS
    return pl.pallas_call(
        _price_body,
        grid=(g,),
        in_specs=[
            pl.BlockSpec((_PR_ROWS, PMID), lambda i: (i, 0)),
            pl.BlockSpec((PMID, N), lambda i: (0, 0)),
            pl.BlockSpec((1, 1), lambda i: (0, 0)),
        ],
        out_specs=pl.BlockSpec((_PR_ROWS, N), lambda i: (i, 0)),
        out_shape=jax.ShapeDtypeStruct((N, N), jnp.float32),
    )(yp, xpt, pr)


def kernel(x, edge_index, conv1_W, conv1_b, lin1_W, lin1_b, lin2_W, lin2_b,
           lin3_W, lin3_b, convp_W, convp_b, lin1p_W, lin1p_b, bilinp_W,
           prices):
    srcb = jnp.pad(edge_index[0], (0, EPAD - E)).reshape(NBPAD, BLK)
    dstb = jnp.pad(edge_index[1], (0, EPAD - E)).reshape(NBPAD, BLK)

    deg_parts = _deg_kernel(dstb)
    d0 = deg_parts[0, :N].reshape(N, 1)
    d1 = deg_parts[1, :N].reshape(N, 1)

    xs = _xs_call(d0, d1, x)

    znd = jnp.zeros((NPAD, D), jnp.float32)
    s_parts = _seg_kernel(srcb, dstb, xs, znd)

    acc, yp, xp = _head_call(
        d0, d1, s_parts, xs, x,
        conv1_W, conv1_b.reshape(1, D),
        lin1_W, lin1_b.reshape(1, MID),
        lin2_W, lin2_b.reshape(1, MID),
        lin3_W, lin3_b.reshape(1, 1),
        convp_W, convp_b.reshape(1, D),
        lin1p_W, lin1p_b.reshape(1, PMID),
        bilinp_W)

    price = _price_call(yp, xp.T, prices.reshape(1, 1))
    return (acc, price)


# speedup vs baseline: 1.1078x; 1.1078x over previous
import functools

import jax
import jax.numpy as jnp
from jax import lax
from jax.experimental import pallas as pl
from jax.experimental.pallas import tpu as pltpu
from jax.experimental.pallas import tpu_sc as plsc

N = 10000
D = 128
E = 320000
NC = 2
NS = 16
NW = NC * NS
BLK = 128
NB = E // BLK
MAXB = 80
NBPAD = NW * MAXB
EPAD = NBPAD * BLK
HB = MAXB // 2
NPAD = 10240
ROWS_T = NPAD // NS
DEGP = 10240
DEG_T = DEGP // NS

_mesh = plsc.VectorSubcoreMesh(core_axis_name="c", subcore_axis_name="s")


def _tile_range(c, s, maxb, num_blocks):
    wid = c * NS + s
    start = maxb * wid
    nb = jnp.clip(num_blocks - start, 0, maxb)
    return start, nb


def _deg_body(dstb, deg_out, dst_v, ones_v, zb_v, acc, sem_a, sem_b):
    c = lax.axis_index("c")
    s = lax.axis_index("s")
    start, nb = _tile_range(c, s, MAXB, NB)

    pltpu.sync_copy(dstb.at[pl.ds(start, MAXB)], dst_v)

    def zero_body(i, _):
        zb_v[pl.ds(i * 16, 16)] = jnp.zeros((16,), jnp.float32)
        return 0

    lax.fori_loop(0, DEG_T // 16, zero_body, 0)
    for j in range(BLK // 16):
        ones_v[pl.ds(j * 16, 16)] = jnp.ones((16,), jnp.float32)
    pltpu.sync_copy(zb_v, acc.at[pl.ds(DEG_T * s, DEG_T)])
    plsc.subcore_barrier()

    def add_start(b, sem):
        pltpu.async_copy(ones_v, acc.at[dst_v.at[b]], sem, add=True)

    def add_wait(b, sem):
        pltpu.make_async_copy(ones_v, acc.at[dst_v.at[b]], sem).wait()

    def body(i, _):
        b0 = 2 * i

        @pl.when(b0 < nb)
        def _():
            @pl.when(b0 >= 2)
            def _():
                add_wait(b0 - 2, sem_a)

            add_start(b0, sem_a)

            @pl.when(b0 >= 2)
            def _():
                add_wait(b0 - 1, sem_b)

            add_start(b0 + 1, sem_b)

        return 0

    lax.fori_loop(0, MAXB // 2, body, 0)
    add_wait(nb - 2, sem_a)
    add_wait(nb - 1, sem_b)
    plsc.subcore_barrier()
    pltpu.sync_copy(acc.at[pl.ds(DEG_T * s, DEG_T)],
                    deg_out.at[c, pl.ds(DEG_T * s, DEG_T)])


_deg_kernel = functools.partial(
    pl.kernel,
    out_type=jax.ShapeDtypeStruct((NC, DEGP), jnp.float32),
    mesh=_mesh,
    scratch_types=[
        pltpu.VMEM((MAXB, BLK), jnp.int32),
        pltpu.VMEM((BLK,), jnp.float32),
        pltpu.VMEM((DEG_T,), jnp.float32),
        pltpu.VMEM_SHARED((DEGP,), jnp.float32),
        pltpu.SemaphoreType.DMA,
        pltpu.SemaphoreType.DMA,
    ],
)(_deg_body)


def _seg_body(srcb, dstb, xs, znd, sum_out,
              src_v, dst_v, rows_a, rows_b, acc,
              sem_a, sem_b, sem_sa, sem_sb):
    c = lax.axis_index("c")
    s = lax.axis_index("s")
    start, nb = _tile_range(c, s, MAXB, NB)

    pltpu.sync_copy(znd.at[pl.ds(ROWS_T * s, ROWS_T)],
                    acc.at[pl.ds(ROWS_T * s, ROWS_T)])
    plsc.subcore_barrier()

    def g_a(b):
        return pltpu.make_async_copy(xs.at[src_v.at[b]], rows_a, sem_a)

    def g_b(b):
        return pltpu.make_async_copy(xs.at[src_v.at[b]], rows_b, sem_b)

    for h in range(MAXB // HB):
        nb_h = jnp.clip(nb - h * HB, 0, HB)

        @pl.when(nb_h > 0)
        def _():
            pltpu.sync_copy(srcb.at[pl.ds(start + h * HB, HB)], src_v)
            pltpu.sync_copy(dstb.at[pl.ds(start + h * HB, HB)], dst_v)
            g_a(0).start()
            g_b(1).start()

            def body(i, _):
                b0 = 2 * i

                @pl.when(b0 < nb_h)
                def _():
                    g_a(b0).wait()
                    pltpu.sync_copy(rows_a, acc.at[dst_v.at[b0]], add=True)

                    @pl.when(b0 + 2 < nb_h)
                    def _():
                        g_a(b0 + 2).start()

                    g_b(b0 + 1).wait()
                    pltpu.sync_copy(rows_b, acc.at[dst_v.at[b0 + 1]], add=True)

                    @pl.when(b0 + 3 < nb_h)
                    def _():
                        g_b(b0 + 3).start()

                return 0

            lax.fori_loop(0, HB // 2, body, 0)

    plsc.subcore_barrier()
    pltpu.sync_copy(acc.at[pl.ds(ROWS_T * s, ROWS_T)],
                    sum_out.at[c, pl.ds(ROWS_T * s, ROWS_T)])


_seg_kernel = functools.partial(
    pl.kernel,
    out_type=jax.ShapeDtypeStruct((NC, NPAD, D), jnp.float32),
    mesh=_mesh,
    scratch_types=[
        pltpu.VMEM((HB, BLK), jnp.int32),
        pltpu.VMEM((HB, BLK), jnp.int32),
        pltpu.VMEM((BLK, D), jnp.float32),
        pltpu.VMEM((BLK, D), jnp.float32),
        pltpu.VMEM_SHARED((NPAD, D), jnp.float32),
        pltpu.SemaphoreType.DMA,
        pltpu.SemaphoreType.DMA,
        pltpu.SemaphoreType.DMA,
        pltpu.SemaphoreType.DMA,
    ],
)(_seg_body)


_XS_ROWS = 2000


def _xs_body(d0_ref, d1_ref, x_ref, o_ref):
    dis = lax.rsqrt(d0_ref[...] + d1_ref[...] + 1.0)
    o_ref[...] = x_ref[...] * dis


def _xs_call(d0, d1, x):
    g = N // _XS_ROWS
    return pl.pallas_call(
        _xs_body,
        grid=(g,),
        in_specs=[
            pl.BlockSpec((_XS_ROWS, 1), lambda i: (i, 0)),
            pl.BlockSpec((_XS_ROWS, 1), lambda i: (i, 0)),
            pl.BlockSpec((_XS_ROWS, D), lambda i: (i, 0)),
        ],
        out_specs=pl.BlockSpec((_XS_ROWS, D), lambda i: (i, 0)),
        out_shape=jax.ShapeDtypeStruct((N, D), jnp.float32),
    )(d0, d1, x)


_HD_ROWS = 2000
MID = 256
PMID = 8


def _dot(a, b):
    return jnp.dot(a, b, preferred_element_type=jnp.float32)


def _head_body(d0_ref, d1_ref, sp_ref, xs_ref, x_ref,
               w1_ref, b1_ref, l1_ref, l1b_ref, l2_ref, l2b_ref,
               l3_ref, l3b_ref, wp_ref, bp_ref, l1p_ref, l1pb_ref, bil_ref,
               acc_ref, yp_ref, xp_ref):
    dis = lax.rsqrt(d0_ref[...] + d1_ref[...] + 1.0)
    x = x_ref[...]
    agg = dis * (sp_ref[0] + sp_ref[1] + xs_ref[...])

    o1 = jax.nn.relu(_dot(agg, w1_ref[...]) + b1_ref[...]) + x
    h = jax.nn.relu(_dot(o1, l1_ref[...]) + l1b_ref[...])
    h = jax.nn.relu(_dot(h, l2_ref[...]) + l2b_ref[...])
    acc_ref[...] = _dot(h, l3_ref[...]) + l3b_ref[...]

    op = jax.nn.relu(_dot(agg, wp_ref[...]) + bp_ref[...]) + x
    xp = jax.nn.relu(_dot(op, l1p_ref[...]) + l1pb_ref[...])
    yp_ref[...] = _dot(xp, bil_ref[...])
    xp_ref[...] = xp


def _head_call(d0, d1, s_parts, xs, x, w1, b1, l1, l1b, l2, l2b, l3, l3b,
               wp, bp, l1p, l1pb, bil):
    g = N // _HD_ROWS
    row = lambda i: (i, 0)
    full = lambda i: (0, 0)
    return pl.pallas_call(
        _head_body,
        grid=(g,),
        in_specs=[
            pl.BlockSpec((_HD_ROWS, 1), row),
            pl.BlockSpec((_HD_ROWS, 1), row),
            pl.BlockSpec((NC, _HD_ROWS, D), lambda i: (0, i, 0)),
            pl.BlockSpec((_HD_ROWS, D), row),
            pl.BlockSpec((_HD_ROWS, D), row),
            pl.BlockSpec((D, D), full),
            pl.BlockSpec((1, D), full),
            pl.BlockSpec((D, MID), full),
            pl.BlockSpec((1, MID), full),
            pl.BlockSpec((MID, MID), full),
            pl.BlockSpec((1, MID), full),
            pl.BlockSpec((MID, 1), full),
            pl.BlockSpec((1, 1), full),
            pl.BlockSpec((D, D), full),
            pl.BlockSpec((1, D), full),
            pl.BlockSpec((D, PMID), full),
            pl.BlockSpec((1, PMID), full),
            pl.BlockSpec((PMID, PMID), full),
        ],
        out_specs=[
            pl.BlockSpec((_HD_ROWS, 1), row),
            pl.BlockSpec((_HD_ROWS, PMID), row),
            pl.BlockSpec((_HD_ROWS, PMID), row),
        ],
        out_shape=[
            jax.ShapeDtypeStruct((N, 1), jnp.float32),
            jax.ShapeDtypeStruct((N, PMID), jnp.float32),
            jax.ShapeDtypeStruct((N, PMID), jnp.float32),
        ],
    )(d0, d1, s_parts, xs, x, w1, b1, l1, l1b, l2, l2b, l3, l3b,
      wp, bp, l1p, l1pb, bil)


_PR_ROWS = 400


def _price_body(yp_ref, xpt_ref, pr_ref, o_ref):
    o_ref[...] = jnp.dot(yp_ref[...], xpt_ref[...],
                         preferred_element_type=jnp.float32) + pr_ref[...]


def _price_call(yp, xpt, pr):
    g = N // _PR_ROWS
    return pl.pallas_call(
        _price_body,
        grid=(g,),
        in_specs=[
            pl.BlockSpec((_PR_ROWS, PMID), lambda i: (i, 0)),
            pl.BlockSpec((PMID, N), lambda i: (0, 0)),
            pl.BlockSpec((1, 1), lambda i: (0, 0)),
        ],
        out_specs=pl.BlockSpec((_PR_ROWS, N), lambda i: (i, 0)),
        out_shape=jax.ShapeDtypeStruct((N, N), jnp.float32),
    )(yp, xpt, pr)


def kernel(x, edge_index, conv1_W, conv1_b, lin1_W, lin1_b, lin2_W, lin2_b,
           lin3_W, lin3_b, convp_W, convp_b, lin1p_W, lin1p_b, bilinp_W,
           prices):
    srcb = jnp.pad(edge_index[0], (0, EPAD - E)).reshape(NBPAD, BLK)
    dstb = jnp.pad(edge_index[1], (0, EPAD - E)).reshape(NBPAD, BLK)

    deg_parts = _deg_kernel(dstb)
    d0 = deg_parts[0, :N].reshape(N, 1)
    d1 = deg_parts[1, :N].reshape(N, 1)

    xs = _xs_call(d0, d1, x)

    znd = jnp.zeros((NPAD, D), jnp.float32)
    s_parts = _seg_kernel(srcb, dstb, xs, znd)

    acc, yp, xp = _head_call(
        d0, d1, s_parts, xs, x,
        conv1_W, conv1_b.reshape(1, D),
        lin1_W, lin1_b.reshape(1, MID),
        lin2_W, lin2_b.reshape(1, MID),
        lin3_W, lin3_b.reshape(1, 1),
        convp_W, convp_b.reshape(1, D),
        lin1p_W, lin1p_b.reshape(1, PMID),
        bilinp_W)

    price = _price_call(yp, xp.T, prices.reshape(1, 1))
    return (acc, price)


# speedup vs baseline: 1.1541x; 1.0418x over previous
import functools

import jax
import jax.numpy as jnp
from jax import lax
from jax.experimental import pallas as pl
from jax.experimental.pallas import tpu as pltpu
from jax.experimental.pallas import tpu_sc as plsc

N = 10000
D = 128
E = 320000
NC = 2
NS = 16
NW = NC * NS
BLK = 128
NB = E // BLK
MAXB = 80
NBPAD = NW * MAXB
EPAD = NBPAD * BLK
HB = MAXB // 2
NPAD = 10240
ROWS_T = NPAD // NS
DEGP = 10240
DEG_T = DEGP // NS

_mesh = plsc.VectorSubcoreMesh(core_axis_name="c", subcore_axis_name="s")


def _tile_range(c, s, maxb, num_blocks):
    wid = c * NS + s
    start = maxb * wid
    nb = jnp.clip(num_blocks - start, 0, maxb)
    return start, nb


def _deg_body(dstb, deg_out, dst_v, ones_v, zb_v, acc, sem_a, sem_b):
    c = lax.axis_index("c")
    s = lax.axis_index("s")
    start, nb = _tile_range(c, s, MAXB, NB)

    pltpu.sync_copy(dstb.at[pl.ds(start, MAXB)], dst_v)

    def zero_body(i, _):
        zb_v[pl.ds(i * 16, 16)] = jnp.zeros((16,), jnp.float32)
        return 0

    lax.fori_loop(0, DEG_T // 16, zero_body, 0)
    for j in range(BLK // 16):
        ones_v[pl.ds(j * 16, 16)] = jnp.ones((16,), jnp.float32)
    pltpu.sync_copy(zb_v, acc.at[pl.ds(DEG_T * s, DEG_T)])
    plsc.subcore_barrier()

    def add_start(b, sem):
        pltpu.async_copy(ones_v, acc.at[dst_v.at[b]], sem, add=True)

    def add_wait(b, sem):
        pltpu.make_async_copy(ones_v, acc.at[dst_v.at[b]], sem).wait()

    def body(i, _):
        b0 = 2 * i

        @pl.when(b0 < nb)
        def _():
            @pl.when(b0 >= 2)
            def _():
                add_wait(b0 - 2, sem_a)

            add_start(b0, sem_a)

            @pl.when(b0 >= 2)
            def _():
                add_wait(b0 - 1, sem_b)

            add_start(b0 + 1, sem_b)

        return 0

    lax.fori_loop(0, MAXB // 2, body, 0)
    add_wait(nb - 2, sem_a)
    add_wait(nb - 1, sem_b)
    plsc.subcore_barrier()
    pltpu.sync_copy(acc.at[pl.ds(DEG_T * s, DEG_T)],
                    deg_out.at[c, pl.ds(DEG_T * s, DEG_T)])


_deg_kernel = functools.partial(
    pl.kernel,
    out_type=jax.ShapeDtypeStruct((NC, DEGP), jnp.float32),
    mesh=_mesh,
    scratch_types=[
        pltpu.VMEM((MAXB, BLK), jnp.int32),
        pltpu.VMEM((BLK,), jnp.float32),
        pltpu.VMEM((DEG_T,), jnp.float32),
        pltpu.VMEM_SHARED((DEGP,), jnp.float32),
        pltpu.SemaphoreType.DMA,
        pltpu.SemaphoreType.DMA,
    ],
)(_deg_body)


def _seg_body(srcb, dstb, xs, znd, sum_out,
              src_v, dst_v, rows_a, rows_b, acc,
              sem_a, sem_b, sem_sa, sem_sb):
    c = lax.axis_index("c")
    s = lax.axis_index("s")
    start, nb = _tile_range(c, s, MAXB, NB)

    def g_a(b):
        return pltpu.make_async_copy(xs.at[src_v.at[b]], rows_a, sem_a)

    def g_b(b):
        return pltpu.make_async_copy(xs.at[src_v.at[b]], rows_b, sem_b)

    zcp = pltpu.make_async_copy(znd.at[pl.ds(ROWS_T * s, ROWS_T)],
                                acc.at[pl.ds(ROWS_T * s, ROWS_T)], sem_sa)
    zcp.start()
    pltpu.sync_copy(srcb.at[pl.ds(start, HB)], src_v)
    pltpu.sync_copy(dstb.at[pl.ds(start, HB)], dst_v)
    g_a(0).start()
    g_b(1).start()
    zcp.wait()
    plsc.subcore_barrier()

    for h in range(MAXB // HB):
        nb_h = jnp.clip(nb - h * HB, 0, HB)

        @pl.when(nb_h > 0)
        def _():
            if h > 0:
                pltpu.sync_copy(srcb.at[pl.ds(start + h * HB, HB)], src_v)
                pltpu.sync_copy(dstb.at[pl.ds(start + h * HB, HB)], dst_v)
                g_a(0).start()
                g_b(1).start()

            def body(i, _):
                b0 = 2 * i

                @pl.when(b0 < nb_h)
                def _():
                    g_a(b0).wait()
                    pltpu.sync_copy(rows_a, acc.at[dst_v.at[b0]], add=True)

                    @pl.when(b0 + 2 < nb_h)
                    def _():
                        g_a(b0 + 2).start()

                    g_b(b0 + 1).wait()
                    pltpu.sync_copy(rows_b, acc.at[dst_v.at[b0 + 1]], add=True)

                    @pl.when(b0 + 3 < nb_h)
                    def _():
                        g_b(b0 + 3).start()

                return 0

            lax.fori_loop(0, HB // 2, body, 0)

    plsc.subcore_barrier()
    pltpu.sync_copy(acc.at[pl.ds(ROWS_T * s, ROWS_T)],
                    sum_out.at[c, pl.ds(ROWS_T * s, ROWS_T)])


_seg_kernel = functools.partial(
    pl.kernel,
    out_type=jax.ShapeDtypeStruct((NC, NPAD, D), jnp.float32),
    mesh=_mesh,
    scratch_types=[
        pltpu.VMEM((HB, BLK), jnp.int32),
        pltpu.VMEM((HB, BLK), jnp.int32),
        pltpu.VMEM((BLK, D), jnp.float32),
        pltpu.VMEM((BLK, D), jnp.float32),
        pltpu.VMEM_SHARED((NPAD, D), jnp.float32),
        pltpu.SemaphoreType.DMA,
        pltpu.SemaphoreType.DMA,
        pltpu.SemaphoreType.DMA,
        pltpu.SemaphoreType.DMA,
    ],
)(_seg_body)


_XS_ROWS = 2000


def _xs_body(d0_ref, d1_ref, x_ref, o_ref):
    dis = lax.rsqrt(d0_ref[...] + d1_ref[...] + 1.0)
    o_ref[...] = x_ref[...] * dis


def _xs_call(d0, d1, x):
    g = N // _XS_ROWS
    return pl.pallas_call(
        _xs_body,
        grid=(g,),
        in_specs=[
            pl.BlockSpec((_XS_ROWS, 1), lambda i: (i, 0)),
            pl.BlockSpec((_XS_ROWS, 1), lambda i: (i, 0)),
            pl.BlockSpec((_XS_ROWS, D), lambda i: (i, 0)),
        ],
        out_specs=pl.BlockSpec((_XS_ROWS, D), lambda i: (i, 0)),
        out_shape=jax.ShapeDtypeStruct((N, D), jnp.float32),
    )(d0, d1, x)


_HD_ROWS = 2000
MID = 256
PMID = 8


def _dot(a, b):
    return jnp.dot(a, b, preferred_element_type=jnp.float32)


_PR_ROWS = 400
_NHD = N // _HD_ROWS
_NPR = N // _PR_ROWS


def _head_body(d0_ref, d1_ref, sp_ref, xs_ref, x_ref,
               w1_ref, b1_ref, l1_ref, l1b_ref, l2_ref, l2b_ref,
               l3_ref, l3b_ref, wp_ref, bp_ref, l1p_ref, l1pb_ref, bil_ref,
               pr_ref, acc_ref, price_ref, yp_s, xpt_s):
    i = pl.program_id(0)

    @pl.when(i < _NHD)
    def _():
        dis = lax.rsqrt(d0_ref[...] + d1_ref[...] + 1.0)
        x = x_ref[...]
        agg = dis * (sp_ref[0] + sp_ref[1] + xs_ref[...])

        o1 = jax.nn.relu(_dot(agg, w1_ref[...]) + b1_ref[...]) + x
        h = jax.nn.relu(_dot(o1, l1_ref[...]) + l1b_ref[...])
        h = jax.nn.relu(_dot(h, l2_ref[...]) + l2b_ref[...])
        acc_ref[...] = _dot(h, l3_ref[...]) + l3b_ref[...]

        op = jax.nn.relu(_dot(agg, wp_ref[...]) + bp_ref[...]) + x
        xp = jax.nn.relu(_dot(op, l1p_ref[...]) + l1pb_ref[...])
        off = pl.multiple_of(i * _HD_ROWS, _HD_ROWS)
        yp_s[pl.ds(off, _HD_ROWS), :] = _dot(xp, bil_ref[...])
        xpt_s[i] = xp.T

    @pl.when(i >= _NHD)
    def _():
        j = i - _NHD
        yp = yp_s[pl.ds(pl.multiple_of(j * _PR_ROWS, _PR_ROWS), _PR_ROWS), :]
        xpt = jnp.concatenate([xpt_s[k] for k in range(_NHD)], axis=1)
        price_ref[...] = jnp.dot(yp, xpt,
                                 preferred_element_type=jnp.float32
                                 ) + pr_ref[...]


def _head_call(d0, d1, s_parts, xs, x, w1, b1, l1, l1b, l2, l2b, l3, l3b,
               wp, bp, l1p, l1pb, bil, pr):
    row = lambda i: (jnp.minimum(i, _NHD - 1), 0)
    full = lambda i: (0, 0)
    return pl.pallas_call(
        _head_body,
        grid=(_NHD + _NPR,),
        in_specs=[
            pl.BlockSpec((_HD_ROWS, 1), row),
            pl.BlockSpec((_HD_ROWS, 1), row),
            pl.BlockSpec((NC, _HD_ROWS, D),
                         lambda i: (0, jnp.minimum(i, _NHD - 1), 0)),
            pl.BlockSpec((_HD_ROWS, D), row),
            pl.BlockSpec((_HD_ROWS, D), row),
            pl.BlockSpec((D, D), full),
            pl.BlockSpec((1, D), full),
            pl.BlockSpec((D, MID), full),
            pl.BlockSpec((1, MID), full),
            pl.BlockSpec((MID, MID), full),
            pl.BlockSpec((1, MID), full),
            pl.BlockSpec((MID, 1), full),
            pl.BlockSpec((1, 1), full),
            pl.BlockSpec((D, D), full),
            pl.BlockSpec((1, D), full),
            pl.BlockSpec((D, PMID), full),
            pl.BlockSpec((1, PMID), full),
            pl.BlockSpec((PMID, PMID), full),
            pl.BlockSpec((1, 1), full),
        ],
        out_specs=[
            pl.BlockSpec((_HD_ROWS, 1), row),
            pl.BlockSpec((_PR_ROWS, N),
                         lambda i: (jnp.maximum(i - _NHD, 0), 0)),
        ],
        out_shape=[
            jax.ShapeDtypeStruct((N, 1), jnp.float32),
            jax.ShapeDtypeStruct((N, N), jnp.float32),
        ],
        scratch_shapes=[
            pltpu.VMEM((N, PMID), jnp.float32),
            pltpu.VMEM((_NHD, PMID, _HD_ROWS), jnp.float32),
        ],
    )(d0, d1, s_parts, xs, x, w1, b1, l1, l1b, l2, l2b, l3, l3b,
      wp, bp, l1p, l1pb, bil, pr)


def kernel(x, edge_index, conv1_W, conv1_b, lin1_W, lin1_b, lin2_W, lin2_b,
           lin3_W, lin3_b, convp_W, convp_b, lin1p_W, lin1p_b, bilinp_W,
           prices):
    srcb = jnp.pad(edge_index[0], (0, EPAD - E)).reshape(NBPAD, BLK)
    dstb = jnp.pad(edge_index[1], (0, EPAD - E)).reshape(NBPAD, BLK)

    deg_parts = _deg_kernel(dstb)
    d0 = deg_parts[0, :N].reshape(N, 1)
    d1 = deg_parts[1, :N].reshape(N, 1)

    xs = _xs_call(d0, d1, x)

    znd = jnp.zeros((NPAD, D), jnp.float32)
    s_parts = _seg_kernel(srcb, dstb, xs, znd)

    acc, price = _head_call(
        d0, d1, s_parts, xs, x,
        conv1_W, conv1_b.reshape(1, D),
        lin1_W, lin1_b.reshape(1, MID),
        lin2_W, lin2_b.reshape(1, MID),
        lin3_W, lin3_b.reshape(1, 1),
        convp_W, convp_b.reshape(1, D),
        lin1p_W, lin1p_b.reshape(1, PMID),
        bilinp_W, prices.reshape(1, 1))
    return (acc, price)
